# baseline (speedup 1.0000x reference)
import functools

import jax
import jax.numpy as jnp
from jax import lax
from jax.experimental import pallas as pl
from jax.experimental.pallas import tpu as pltpu
from jax.experimental.pallas import tpu_sc as plsc

HID = 2048
MEM = 50000
TOPK = 64
BM = 400
NW = 32
ROWS_W = 400
R_SC = NW * ROWS_W
S_TC = MEM - R_SC
NB_TC = S_TC // BM
NB = MEM // BM
CH = 16
NCH = ROWS_W // CH
BQ = 256
NCAND = 96
NEG = float("-inf")


def _q_body(query_ref, wq_ref, bq_ref, q_ref):
    q_ref[...] = lax.dot_general(
        query_ref[...], wq_ref[...],
        (((1,), (1,)), ((), ())),
        preferred_element_type=jnp.float32,
    ) + bq_ref[...]


def _stream_body(q_ref, keys_ref, imp_ref, age_ref, l_ref):
    logits = lax.dot_general(
        q_ref[...], keys_ref[...],
        (((1,), (1,)), ((), ())),
        preferred_element_type=jnp.float32,
    )
    imp = imp_ref[...].reshape(1, BM)
    age = age_ref[...].reshape(1, BM)
    l_ref[...] = (logits * imp / (1.0 + age)).reshape(1, 1, BM)


def _sc_body(keys_hbm, q_hbm, out_hbm, q_v, kb0, kb1, dots_v, sem0, sem1):
    wid = lax.axis_index("s") * 2 + lax.axis_index("c")
    rowbase = S_TC + wid * ROWS_W
    pltpu.sync_copy(q_hbm, q_v)
    io16 = lax.iota(jnp.int32, 16)

    def bf16r(x):
        u = plsc.bitcast(x, jnp.uint32)
        r = (u + jnp.uint32(0x8000)) & jnp.uint32(0xFFFF0000)
        return plsc.bitcast(r, jnp.float32)

    def qround(c, _):
        q_v[0, pl.ds(c * 16, 16)] = bf16r(q_v[0, pl.ds(c * 16, 16)])
        return 0

    lax.fori_loop(0, HID // 16, qround, 0)

    def start(ch, buf, sem):
        pltpu.make_async_copy(
            keys_hbm.at[pl.ds(rowbase + ch * CH, CH), :],
            buf, sem).start()

    def compute(ch, buf, sem):
        pltpu.make_async_copy(
            keys_hbm.at[pl.ds(0, CH), :], buf, sem).wait()

        def cbody(c, accs):
            for u in range(8):
                off = (c * 8 + u) * 16
                qv = q_v[0, pl.ds(off, 16)]
                accs = tuple(
                    accs[r] + bf16r(buf[r, pl.ds(off, 16)]) * qv
                    for r in range(CH))
            return accs

        accs = lax.fori_loop(
            0, HID // 128, cbody,
            tuple(jnp.zeros((16,), jnp.float32) for _ in range(CH)))
        res = jnp.zeros((16,), jnp.float32)
        for r in range(CH):
            res = jnp.where(io16 == r, jnp.sum(accs[r]), res)
        dots_v[pl.ds(ch * CH, CH)] = res

    start(0, kb0, sem0)

    def pair(g, _):
        start(2 * g + 1, kb1, sem1)
        compute(2 * g, kb0, sem0)
        start(2 * g + 2, kb0, sem0)
        compute(2 * g + 1, kb1, sem1)
        return 0

    lax.fori_loop(0, (NCH - 1) // 2, pair, 0)
    compute(NCH - 1, kb0, sem0)
    pltpu.sync_copy(dots_v, out_hbm.at[pl.ds(wid * ROWS_W, ROWS_W)])


def _make_sc_call():
    return pl.kernel(
        _sc_body,
        out_type=jax.ShapeDtypeStruct((R_SC,), jnp.float32),
        mesh=plsc.VectorSubcoreMesh(
            core_axis_name="c", subcore_axis_name="s",
            num_cores=2, num_subcores=16),
        scratch_types=[
            pltpu.VMEM((1, HID), jnp.float32),
            pltpu.VMEM((CH, HID), jnp.float32),
            pltpu.VMEM((CH, HID), jnp.float32),
            pltpu.VMEM((ROWS_W,), jnp.float32),
            pltpu.SemaphoreType.DMA,
            pltpu.SemaphoreType.DMA,
        ],
        compiler_params=pltpu.CompilerParams(needs_layout_passes=False),
    )


def _merge_body(q_ref, ltc_ref, scd_ref, imp_ref, age_ref, mk_ref, mv_ref,
                recalled_ref, values_ref,
                l_ref, scale_ref, kr_ref, vr_ref, semk, semv):
    scale = imp_ref[...] / (1.0 + age_ref[...])
    scale_ref[...] = scale
    l_ref[0:NB_TC, :] = ltc_ref[...].reshape(NB_TC, BM)
    l_ref[NB_TC:NB, :] = scd_ref[...] * scale[NB_TC:NB, :]

    lfull = l_ref[...]
    mxcol = jnp.max(lfull, axis=1, keepdims=True)
    lio = lax.broadcasted_iota(jnp.int32, (1, 128), 1)
    cio = lax.broadcasted_iota(jnp.int32, (1, BM), 1)
    kio = lax.broadcasted_iota(jnp.int32, (1, TOPK), 1)
    nio = lax.broadcasted_iota(jnp.int32, (1, NCAND), 1)
    mxt = lax.transpose(mxcol, (1, 0))
    mx0 = jnp.concatenate(
        [mxt, jnp.full((1, 128 - NB), NEG, jnp.float32)], axis=1)
    gmax = jnp.max(mx0)
    denom = jnp.sum(jnp.exp(lfull - gmax))

    def pick(j, carry):
        mx, fvec, svec = carry
        gm = jnp.max(mx)
        ridx = jnp.min(jnp.where(mx == gm, lio, NB))
        row = l_ref[pl.ds(ridx, 1), :]
        cidx = jnp.min(jnp.where(row == gm, cio, BM))
        flat = ridx * BM + cidx
        pltpu.make_async_copy(
            mk_ref.at[pl.ds(flat, 1), :],
            kr_ref.at[pl.ds(j, 1), :], semk).start()
        srow = scale_ref[pl.ds(ridx, 1), :]
        sc_j = jnp.max(jnp.where(cio == cidx, srow, NEG))
        newrow = jnp.where(cio == cidx, NEG, row)
        l_ref[pl.ds(ridx, 1), :] = newrow
        mx = jnp.where(lio == ridx, jnp.max(newrow), mx)
        fvec = jnp.where(nio == j, flat, fvec)
        svec = jnp.where(nio == j, sc_j, svec)
        return mx, fvec, svec

    _, fvec, svec = lax.fori_loop(
        0, NCAND, pick,
        (mx0, jnp.zeros((1, NCAND), jnp.int32),
         jnp.zeros((1, NCAND), jnp.float32)))

    def draink(j, _):
        pltpu.make_async_copy(
            mk_ref.at[pl.ds(0, 1), :],
            kr_ref.at[pl.ds(0, 1), :], semk).wait()
        return 0

    lax.fori_loop(0, NCAND, draink, 0)

    sel = lax.dot_general(
        q_ref[...], kr_ref[...],
        (((1,), (1,)), ((), ())),
        preferred_element_type=jnp.float32,
    ) * svec

    def pick2(m, carry):
        sel, vals = carry
        gm2 = jnp.max(sel)
        c2 = jnp.min(jnp.where(sel == gm2, nio, NCAND))
        flat2 = jnp.max(jnp.where(nio == c2, fvec, -1))
        pltpu.make_async_copy(
            mv_ref.at[pl.ds(flat2, 1), :],
            vr_ref.at[pl.ds(m, 1), :], semv).start()
        sel = jnp.where(nio == c2, NEG, sel)
        vals = jnp.where(kio == m, gm2, vals)
        return sel, vals

    _, vals = lax.fori_loop(
        0, TOPK, pick2,
        (sel, jnp.full((1, TOPK), NEG, jnp.float32)))

    def drainv(m, _):
        pltpu.make_async_copy(
            mv_ref.at[pl.ds(0, 1), :],
            vr_ref.at[pl.ds(0, 1), :], semv).wait()
        return 0

    lax.fori_loop(0, TOPK, drainv, 0)

    w = jnp.exp(vals - gmax) / denom
    values_ref[...] = w
    recalled_ref[...] = lax.dot_general(
        w, vr_ref[...],
        (((1,), (0,)), ((), ())),
        preferred_element_type=jnp.float32,
    )


def kernel(query, Wq, bq, memory_keys, memory_values, memory_importance,
           memory_age, top_k):
    del top_k
    query2 = query.reshape(1, HID)
    bq2 = bq.reshape(1, HID)
    imp3 = memory_importance[:S_TC].reshape(NB_TC, 1, BM)
    age3 = memory_age[:S_TC].reshape(NB_TC, 1, BM)
    imp2 = memory_importance.reshape(NB, BM)
    age2 = memory_age.reshape(NB, BM)

    q = pl.pallas_call(
        _q_body,
        grid=(HID // BQ,),
        in_specs=[
            pl.BlockSpec((1, HID), lambda i: (0, 0)),
            pl.BlockSpec((BQ, HID), lambda i: (i, 0)),
            pl.BlockSpec((1, BQ), lambda i: (0, i)),
        ],
        out_specs=pl.BlockSpec((1, BQ), lambda i: (0, i)),
        out_shape=jax.ShapeDtypeStruct((1, HID), jnp.float32),
    )(query2, Wq, bq2)

    sc_dots = _make_sc_call()(memory_keys, q)

    l_tc = pl.pallas_call(
        _stream_body,
        grid=(NB_TC,),
        in_specs=[
            pl.BlockSpec((1, HID), lambda i: (0, 0)),
            pl.BlockSpec((BM, HID), lambda i: (i, 0)),
            pl.BlockSpec((1, 1, BM), lambda i: (i, 0, 0)),
            pl.BlockSpec((1, 1, BM), lambda i: (i, 0, 0)),
        ],
        out_specs=pl.BlockSpec((1, 1, BM), lambda i: (i, 0, 0)),
        out_shape=jax.ShapeDtypeStruct((NB_TC, 1, BM), jnp.float32),
    )(q, memory_keys, imp3, age3)

    recalled, values = pl.pallas_call(
        _merge_body,
        in_specs=[
            pl.BlockSpec((1, HID), lambda: (0, 0)),
            pl.BlockSpec((NB_TC, 1, BM), lambda: (0, 0, 0)),
            pl.BlockSpec((NW, ROWS_W), lambda: (0, 0)),
            pl.BlockSpec((NB, BM), lambda: (0, 0)),
            pl.BlockSpec((NB, BM), lambda: (0, 0)),
            pl.BlockSpec(memory_space=pltpu.MemorySpace.HBM),
            pl.BlockSpec(memory_space=pltpu.MemorySpace.HBM),
        ],
        out_specs=[
            pl.BlockSpec((1, HID), lambda: (0, 0)),
            pl.BlockSpec((1, TOPK), lambda: (0, 0)),
        ],
        out_shape=[
            jax.ShapeDtypeStruct((1, HID), jnp.float32),
            jax.ShapeDtypeStruct((1, TOPK), jnp.float32),
        ],
        scratch_shapes=[
            pltpu.VMEM((NB, BM), jnp.float32),
            pltpu.VMEM((NB, BM), jnp.float32),
            pltpu.VMEM((NCAND, HID), jnp.float32),
            pltpu.VMEM((TOPK, HID), jnp.float32),
            pltpu.SemaphoreType.DMA,
            pltpu.SemaphoreType.DMA,
        ],
    )(q, l_tc, sc_dots.reshape(NW, ROWS_W), imp2, age2, memory_keys,
      memory_values)

    return recalled.reshape(HID), values.reshape(TOPK)


# speedup vs baseline: 1.0728x; 1.0728x over previous
import functools

import jax
import jax.numpy as jnp
from jax import lax
from jax.experimental import pallas as pl
from jax.experimental.pallas import tpu as pltpu
from jax.experimental.pallas import tpu_sc as plsc

HID = 2048
MEM = 50000
TOPK = 64
BM = 400
NW = 32
ROWS_W = 400
R_SC = NW * ROWS_W
S_TC = MEM - R_SC
NB_TC = S_TC // BM
NB = MEM // BM
CH = 16
NCH = ROWS_W // CH
BQ = 256
BMS = 2480
NBS = S_TC // BMS
NCAND = 96
NEG = float("-inf")


def _q_body(query_ref, wq_ref, bq_ref, q_ref):
    q_ref[...] = lax.dot_general(
        query_ref[...], wq_ref[...],
        (((1,), (1,)), ((), ())),
        preferred_element_type=jnp.float32,
    ) + bq_ref[...]


def _stream_body(q_ref, keys_ref, imp_ref, age_ref, l_ref):
    logits = lax.dot_general(
        q_ref[...], keys_ref[...],
        (((1,), (1,)), ((), ())),
        preferred_element_type=jnp.float32,
    )
    imp = imp_ref[...].reshape(1, BMS)
    age = age_ref[...].reshape(1, BMS)
    l_ref[...] = (logits * imp / (1.0 + age)).reshape(1, 1, BMS)


def _sc_body(keys_hbm, q_hbm, out_hbm, q_v, kb0, kb1, dots_v, sem0, sem1):
    wid = lax.axis_index("s") * 2 + lax.axis_index("c")
    rowbase = S_TC + wid * ROWS_W
    pltpu.sync_copy(q_hbm, q_v)
    io16 = lax.iota(jnp.int32, 16)

    def bf16r(x):
        u = plsc.bitcast(x, jnp.uint32)
        r = (u + jnp.uint32(0x8000)) & jnp.uint32(0xFFFF0000)
        return plsc.bitcast(r, jnp.float32)

    def qround(c, _):
        q_v[0, pl.ds(c * 16, 16)] = bf16r(q_v[0, pl.ds(c * 16, 16)])
        return 0

    lax.fori_loop(0, HID // 16, qround, 0)

    def start(ch, buf, sem):
        pltpu.make_async_copy(
            keys_hbm.at[pl.ds(rowbase + ch * CH, CH), :],
            buf, sem).start()

    def compute(ch, buf, sem):
        pltpu.make_async_copy(
            keys_hbm.at[pl.ds(0, CH), :], buf, sem).wait()

        def cbody(c, accs):
            for u in range(8):
                off = (c * 8 + u) * 16
                qv = q_v[0, pl.ds(off, 16)]
                accs = tuple(
                    accs[r] + bf16r(buf[r, pl.ds(off, 16)]) * qv
                    for r in range(CH))
            return accs

        accs = lax.fori_loop(
            0, HID // 128, cbody,
            tuple(jnp.zeros((16,), jnp.float32) for _ in range(CH)))
        res = jnp.zeros((16,), jnp.float32)
        for r in range(CH):
            res = jnp.where(io16 == r, jnp.sum(accs[r]), res)
        dots_v[pl.ds(ch * CH, CH)] = res

    start(0, kb0, sem0)

    def pair(g, _):
        start(2 * g + 1, kb1, sem1)
        compute(2 * g, kb0, sem0)
        start(2 * g + 2, kb0, sem0)
        compute(2 * g + 1, kb1, sem1)
        return 0

    lax.fori_loop(0, (NCH - 1) // 2, pair, 0)
    compute(NCH - 1, kb0, sem0)
    pltpu.sync_copy(dots_v, out_hbm.at[pl.ds(wid * ROWS_W, ROWS_W)])


def _make_sc_call():
    return pl.kernel(
        _sc_body,
        out_type=jax.ShapeDtypeStruct((R_SC,), jnp.float32),
        mesh=plsc.VectorSubcoreMesh(
            core_axis_name="c", subcore_axis_name="s",
            num_cores=2, num_subcores=16),
        scratch_types=[
            pltpu.VMEM((1, HID), jnp.float32),
            pltpu.VMEM((CH, HID), jnp.float32),
            pltpu.VMEM((CH, HID), jnp.float32),
            pltpu.VMEM((ROWS_W,), jnp.float32),
            pltpu.SemaphoreType.DMA,
            pltpu.SemaphoreType.DMA,
        ],
        compiler_params=pltpu.CompilerParams(needs_layout_passes=False),
    )


def _merge_body(q_ref, ltc_ref, scd_ref, imp_ref, age_ref, mk_ref, mv_ref,
                recalled_ref, values_ref,
                l_ref, scale_ref, kr_ref, vr_ref, semk, semv):
    scale = imp_ref[...] / (1.0 + age_ref[...])
    scale_ref[...] = scale
    l_ref[0:NB_TC, :] = ltc_ref[...].reshape(NB_TC, BM)
    l_ref[NB_TC:NB, :] = scd_ref[...] * scale[NB_TC:NB, :]

    lfull = l_ref[...]
    mxcol = jnp.max(lfull, axis=1, keepdims=True)
    lio = lax.broadcasted_iota(jnp.int32, (1, 128), 1)
    cio = lax.broadcasted_iota(jnp.int32, (1, BM), 1)
    kio = lax.broadcasted_iota(jnp.int32, (1, TOPK), 1)
    nio = lax.broadcasted_iota(jnp.int32, (1, NCAND), 1)
    mxt = lax.transpose(mxcol, (1, 0))
    mx0 = jnp.concatenate(
        [mxt, jnp.full((1, 128 - NB), NEG, jnp.float32)], axis=1)
    gmax = jnp.max(mx0)
    denom = jnp.sum(jnp.exp(lfull - gmax))

    def pick(j, carry):
        mx, fvec, svec = carry
        gm = jnp.max(mx)
        ridx = jnp.min(jnp.where(mx == gm, lio, NB))
        row = l_ref[pl.ds(ridx, 1), :]
        cidx = jnp.min(jnp.where(row == gm, cio, BM))
        flat = ridx * BM + cidx
        pltpu.make_async_copy(
            mk_ref.at[pl.ds(flat, 1), :],
            kr_ref.at[pl.ds(j, 1), :], semk).start()
        srow = scale_ref[pl.ds(ridx, 1), :]
        sc_j = jnp.max(jnp.where(cio == cidx, srow, NEG))
        newrow = jnp.where(cio == cidx, NEG, row)
        l_ref[pl.ds(ridx, 1), :] = newrow
        mx = jnp.where(lio == ridx, jnp.max(newrow), mx)
        fvec = jnp.where(nio == j, flat, fvec)
        svec = jnp.where(nio == j, sc_j, svec)
        return mx, fvec, svec

    _, fvec, svec = lax.fori_loop(
        0, NCAND, pick,
        (mx0, jnp.zeros((1, NCAND), jnp.int32),
         jnp.zeros((1, NCAND), jnp.float32)))

    def draink(j, _):
        pltpu.make_async_copy(
            mk_ref.at[pl.ds(0, 1), :],
            kr_ref.at[pl.ds(0, 1), :], semk).wait()
        return 0

    lax.fori_loop(0, NCAND, draink, 0)

    sel = lax.dot_general(
        q_ref[...], kr_ref[...],
        (((1,), (1,)), ((), ())),
        preferred_element_type=jnp.float32,
    ) * svec

    def pick2(m, carry):
        sel, vals = carry
        gm2 = jnp.max(sel)
        c2 = jnp.min(jnp.where(sel == gm2, nio, NCAND))
        flat2 = jnp.max(jnp.where(nio == c2, fvec, -1))
        pltpu.make_async_copy(
            mv_ref.at[pl.ds(flat2, 1), :],
            vr_ref.at[pl.ds(m, 1), :], semv).start()
        sel = jnp.where(nio == c2, NEG, sel)
        vals = jnp.where(kio == m, gm2, vals)
        return sel, vals

    _, vals = lax.fori_loop(
        0, TOPK, pick2,
        (sel, jnp.full((1, TOPK), NEG, jnp.float32)))

    def drainv(m, _):
        pltpu.make_async_copy(
            mv_ref.at[pl.ds(0, 1), :],
            vr_ref.at[pl.ds(0, 1), :], semv).wait()
        return 0

    lax.fori_loop(0, TOPK, drainv, 0)

    w = jnp.exp(vals - gmax) / denom
    values_ref[...] = w
    recalled_ref[...] = lax.dot_general(
        w, vr_ref[...],
        (((1,), (0,)), ((), ())),
        preferred_element_type=jnp.float32,
    )


def kernel(query, Wq, bq, memory_keys, memory_values, memory_importance,
           memory_age, top_k):
    del top_k
    query2 = query.reshape(1, HID)
    bq2 = bq.reshape(1, HID)
    imp3 = memory_importance[:S_TC].reshape(NBS, 1, BMS)
    age3 = memory_age[:S_TC].reshape(NBS, 1, BMS)
    imp2 = memory_importance.reshape(NB, BM)
    age2 = memory_age.reshape(NB, BM)

    q = pl.pallas_call(
        _q_body,
        grid=(HID // BQ,),
        in_specs=[
            pl.BlockSpec((1, HID), lambda i: (0, 0)),
            pl.BlockSpec((BQ, HID), lambda i: (i, 0)),
            pl.BlockSpec((1, BQ), lambda i: (0, i)),
        ],
        out_specs=pl.BlockSpec((1, BQ), lambda i: (0, i)),
        out_shape=jax.ShapeDtypeStruct((1, HID), jnp.float32),
    )(query2, Wq, bq2)

    sc_dots = _make_sc_call()(memory_keys, q)

    l_tc = pl.pallas_call(
        _stream_body,
        grid=(NBS,),
        in_specs=[
            pl.BlockSpec((1, HID), lambda i: (0, 0)),
            pl.BlockSpec((BMS, HID), lambda i: (i, 0)),
            pl.BlockSpec((1, 1, BMS), lambda i: (i, 0, 0)),
            pl.BlockSpec((1, 1, BMS), lambda i: (i, 0, 0)),
        ],
        out_specs=pl.BlockSpec((1, 1, BMS), lambda i: (i, 0, 0)),
        out_shape=jax.ShapeDtypeStruct((NBS, 1, BMS), jnp.float32),
    )(q, memory_keys, imp3, age3)
    l_tc = l_tc.reshape(NB_TC, 1, BM)

    recalled, values = pl.pallas_call(
        _merge_body,
        in_specs=[
            pl.BlockSpec((1, HID), lambda: (0, 0)),
            pl.BlockSpec((NB_TC, 1, BM), lambda: (0, 0, 0)),
            pl.BlockSpec((NW, ROWS_W), lambda: (0, 0)),
            pl.BlockSpec((NB, BM), lambda: (0, 0)),
            pl.BlockSpec((NB, BM), lambda: (0, 0)),
            pl.BlockSpec(memory_space=pltpu.MemorySpace.HBM),
            pl.BlockSpec(memory_space=pltpu.MemorySpace.HBM),
        ],
        out_specs=[
            pl.BlockSpec((1, HID), lambda: (0, 0)),
            pl.BlockSpec((1, TOPK), lambda: (0, 0)),
        ],
        out_shape=[
            jax.ShapeDtypeStruct((1, HID), jnp.float32),
            jax.ShapeDtypeStruct((1, TOPK), jnp.float32),
        ],
        scratch_shapes=[
            pltpu.VMEM((NB, BM), jnp.float32),
            pltpu.VMEM((NB, BM), jnp.float32),
            pltpu.VMEM((NCAND, HID), jnp.float32),
            pltpu.VMEM((TOPK, HID), jnp.float32),
            pltpu.SemaphoreType.DMA,
            pltpu.SemaphoreType.DMA,
        ],
    )(q, l_tc, sc_dots.reshape(NW, ROWS_W), imp2, age2, memory_keys,
      memory_values)

    return recalled.reshape(HID), values.reshape(TOPK)
